# Initial kernel scaffold; baseline (speedup 1.0000x reference)
import jax
import jax.numpy as jnp
from jax.experimental import pallas as pl


def kernel(mh_output, W_route, b_route, W_noise, b_noise):
    raise NotImplementedError("write your pallas kernel here")


# speedup vs baseline: 4.4167x; 4.4167x over previous
import jax
import jax.numpy as jnp
from jax.experimental import pallas as pl

_B, _L, _D, _E, _TOP_K = 4, 2048, 4096, 64, 8
_TL = 256


def _router_kernel(x_ref, w_ref, b_ref, noise_ref, out_ref, idx_ref):
    x = x_ref[...].reshape(_B * _TL, _D)
    y = jnp.dot(x, w_ref[...], preferred_element_type=jnp.float32) + b_ref[...]
    logits = y[:, :_E]
    noise_logits = y[:, _E:]
    noisy = logits + noise_ref[...].reshape(_B * _TL, _E) * jax.nn.softplus(noise_logits)
    noisy3 = noisy.reshape(_B, _TL, _E)
    mean = jnp.mean(noisy3, axis=0)

    iota = jax.lax.broadcasted_iota(jnp.int32, (_TL, _E), 1)
    work = mean
    mask = jnp.zeros((_TL, _E), dtype=jnp.bool_)
    cols = []
    for _ in range(_TOP_K):
        m = jnp.max(work, axis=1, keepdims=True)
        sel = jnp.min(jnp.where(work == m, iota, _E), axis=1, keepdims=True)
        hit = iota == sel
        mask = mask | hit
        work = jnp.where(hit, -jnp.inf, work)
        cols.append(sel)
    idx_ref[...] = jnp.concatenate(cols, axis=1)

    masked = jnp.where(mask[None], noisy3, -jnp.inf)
    out_ref[...] = jax.nn.softmax(masked, axis=-1)


def kernel(mh_output, W_route, b_route, W_noise, b_noise):
    W = jnp.concatenate([W_route, W_noise], axis=0).T
    bias = jnp.concatenate([b_route, b_noise]).reshape(1, 2 * _E)
    noise = jax.random.normal(jax.random.key(42), (_B, _L, _E), dtype=jnp.float32)

    grid = (_L // _TL,)
    router_output, indices = pl.pallas_call(
        _router_kernel,
        grid=grid,
        in_specs=[
            pl.BlockSpec((_B, _TL, _D), lambda i: (0, i, 0)),
            pl.BlockSpec((_D, 2 * _E), lambda i: (0, 0)),
            pl.BlockSpec((1, 2 * _E), lambda i: (0, 0)),
            pl.BlockSpec((_B, _TL, _E), lambda i: (0, i, 0)),
        ],
        out_specs=[
            pl.BlockSpec((_B, _TL, _E), lambda i: (0, i, 0)),
            pl.BlockSpec((_TL, _TOP_K), lambda i: (i, 0)),
        ],
        out_shape=[
            jax.ShapeDtypeStruct((_B, _L, _E), jnp.float32),
            jax.ShapeDtypeStruct((_L, _TOP_K), jnp.int32),
        ],
    )(mh_output, W, bias, noise)

    return router_output, jnp.broadcast_to(indices[None], (_B, _L, _TOP_K))


# speedup vs baseline: 4.4435x; 1.0061x over previous
import jax
import jax.numpy as jnp
from jax.experimental import pallas as pl

_B, _L, _D, _E, _TOP_K = 4, 2048, 4096, 64, 8
_TL = 256


def _router_kernel(x_ref, w_ref, b_ref, noise_ref, out_ref, idx_ref):
    x = x_ref[...].reshape(_B * _TL, _D)
    y = jax.lax.dot_general(
        x, w_ref[...], (((1,), (1,)), ((), ())),
        preferred_element_type=jnp.float32) + b_ref[...]
    logits = y[:, :_E]
    noise_logits = y[:, _E:]
    noisy = logits + noise_ref[...].reshape(_B * _TL, _E) * jax.nn.softplus(noise_logits)
    noisy3 = noisy.reshape(_B, _TL, _E)
    mean = jnp.mean(noisy3, axis=0)

    iota = jax.lax.broadcasted_iota(jnp.int32, (_TL, _E), 1)
    work = mean
    mask = jnp.zeros((_TL, _E), dtype=jnp.bool_)
    cols = []
    for _ in range(_TOP_K):
        m = jnp.max(work, axis=1, keepdims=True)
        sel = jnp.min(jnp.where(work == m, iota, _E), axis=1, keepdims=True)
        hit = iota == sel
        mask = mask | hit
        work = jnp.where(hit, -jnp.inf, work)
        cols.append(sel)
    idx = jnp.concatenate(cols, axis=1)
    idx_ref[...] = jnp.broadcast_to(idx[None], (_B, _TL, _TOP_K))

    masked = jnp.where(mask[None], noisy3, -jnp.inf)
    out_ref[...] = jax.nn.softmax(masked, axis=-1)


def kernel(mh_output, W_route, b_route, W_noise, b_noise):
    W = jnp.concatenate([W_route, W_noise], axis=0)
    bias = jnp.concatenate([b_route, b_noise]).reshape(1, 2 * _E)
    noise = jax.random.normal(jax.random.key(42), (_B, _L, _E), dtype=jnp.float32)

    grid = (_L // _TL,)
    router_output, indices = pl.pallas_call(
        _router_kernel,
        grid=grid,
        in_specs=[
            pl.BlockSpec((_B, _TL, _D), lambda i: (0, i, 0)),
            pl.BlockSpec((2 * _E, _D), lambda i: (0, 0)),
            pl.BlockSpec((1, 2 * _E), lambda i: (0, 0)),
            pl.BlockSpec((_B, _TL, _E), lambda i: (0, i, 0)),
        ],
        out_specs=[
            pl.BlockSpec((_B, _TL, _E), lambda i: (0, i, 0)),
            pl.BlockSpec((_B, _TL, _TOP_K), lambda i: (0, i, 0)),
        ],
        out_shape=[
            jax.ShapeDtypeStruct((_B, _L, _E), jnp.float32),
            jax.ShapeDtypeStruct((_B, _L, _TOP_K), jnp.int32),
        ],
    )(mh_output, W, bias, noise)

    return router_output, indices
